# speedup vs baseline: 2.7506x; 2.7506x over previous
import functools

import jax
import jax.numpy as jnp
from jax import lax
from jax.experimental import pallas as pl
from jax.experimental.pallas import tpu as pltpu
from jax.experimental.pallas import tpu_sc as plsc

_D = 128
_B = 4096 * 50
_NW = 32
_BPW = _B // _NW
_CHUNK = 128
_NCHUNK = _BPW // _CHUNK


def _emb_body(idx_hbm, table_hbm, out_hbm, idx_v, rows_v, sem):
    wid = lax.axis_index("s") * 2 + lax.axis_index("c")
    base = wid * _BPW

    def step(g, carry):
        off = base + g * _CHUNK
        pltpu.sync_copy(idx_hbm.at[pl.ds(off, _CHUNK)], idx_v)
        pltpu.async_copy(table_hbm.at[idx_v], rows_v, sem).wait()
        pltpu.sync_copy(rows_v, out_hbm.at[pl.ds(off, _CHUNK)])
        return carry

    lax.fori_loop(0, _NCHUNK, step, 0)


_emb = functools.partial(
    pl.kernel,
    mesh=plsc.VectorSubcoreMesh(core_axis_name="c", subcore_axis_name="s"),
    out_type=jax.ShapeDtypeStruct((_B, _D), jnp.float32),
    scratch_types=[
        pltpu.VMEM((_CHUNK,), jnp.int32),
        pltpu.VMEM((_CHUNK, _D), jnp.float32),
        pltpu.SemaphoreType.DMA,
    ],
)(_emb_body)


@jax.jit
def kernel(token_ids, weight):
    flat = token_ids.reshape(-1).astype(jnp.int32)
    out = _emb(flat, weight)
    return out.reshape(*token_ids.shape, _D)


# speedup vs baseline: 3.1105x; 1.1308x over previous
import functools

import jax
import jax.numpy as jnp
from jax import lax
from jax.experimental import pallas as pl
from jax.experimental.pallas import tpu as pltpu
from jax.experimental.pallas import tpu_sc as plsc

_D = 128
_B = 4096 * 50
_NW = 32
_BPW = _B // _NW
_CHUNK = 128
_NCHUNK = _BPW // _CHUNK


def _emb_body(idx_hbm, table_hbm, out_hbm, idx_v, rows_v, gs0, gs1, os0, os1):
    wid = lax.axis_index("s") * 2 + lax.axis_index("c")
    base = wid * _BPW

    pltpu.sync_copy(idx_hbm.at[wid], idx_v)

    def gather(g, buf, sem):
        pltpu.async_copy(table_hbm.at[idx_v.at[g]], buf, sem)

    def gwait(g, buf, sem):
        pltpu.make_async_copy(table_hbm.at[idx_v.at[g]], buf, sem).wait()

    def store(g, buf, sem):
        pltpu.async_copy(buf, out_hbm.at[pl.ds(base + g * _CHUNK, _CHUNK)], sem)

    def swait(g, buf, sem):
        pltpu.make_async_copy(
            buf, out_hbm.at[pl.ds(base + g * _CHUNK, _CHUNK)], sem
        ).wait()

    b0 = rows_v.at[0]
    b1 = rows_v.at[1]

    gather(0, b0, gs0)
    gwait(0, b0, gs0)
    store(0, b0, os0)
    gather(1, b1, gs1)
    gwait(1, b1, gs1)
    store(1, b1, os1)
    swait(0, b0, os0)
    gather(2, b0, gs0)

    def body(t, carry):
        gwait(2 * t, b0, gs0)
        store(2 * t, b0, os0)
        swait(2 * t - 1, b1, os1)
        gather(2 * t + 1, b1, gs1)
        gwait(2 * t + 1, b1, gs1)
        store(2 * t + 1, b1, os1)
        swait(2 * t, b0, os0)
        gather(jnp.minimum(2 * t + 2, _NCHUNK - 1), b0, gs0)
        return carry

    lax.fori_loop(1, _NCHUNK // 2, body, 0)

    gwait(_NCHUNK - 1, b0, gs0)
    swait(_NCHUNK - 1, b1, os1)


_emb = functools.partial(
    pl.kernel,
    mesh=plsc.VectorSubcoreMesh(core_axis_name="c", subcore_axis_name="s"),
    out_type=jax.ShapeDtypeStruct((_B, _D), jnp.float32),
    scratch_types=[
        pltpu.VMEM((_NCHUNK, _CHUNK), jnp.int32),
        pltpu.VMEM((2, _CHUNK, _D), jnp.float32),
        pltpu.SemaphoreType.DMA,
        pltpu.SemaphoreType.DMA,
        pltpu.SemaphoreType.DMA,
        pltpu.SemaphoreType.DMA,
    ],
)(_emb_body)


@jax.jit
def kernel(token_ids, weight):
    flat = token_ids.reshape(_NW, _NCHUNK, _CHUNK).astype(jnp.int32)
    out = _emb(flat, weight)
    return out.reshape(*token_ids.shape, _D)


# speedup vs baseline: 3.3269x; 1.0696x over previous
import functools

import jax
import jax.numpy as jnp
from jax import lax
from jax.experimental import pallas as pl
from jax.experimental.pallas import tpu as pltpu
from jax.experimental.pallas import tpu_sc as plsc

_D = 128
_B = 4096 * 50
_NW = 32
_BPW = _B // _NW
_CHUNK = 128
_NCHUNK = _BPW // _CHUNK
_NBUF = 5
_LOOK = 3


def _emb_body(idx_hbm, table_hbm, out_hbm, idx_v, rows_v, *sems):
    gs = sems[:_NBUF]
    ss = sems[_NBUF:]
    wid = lax.axis_index("s") * 2 + lax.axis_index("c")
    base = wid * _BPW

    pltpu.sync_copy(idx_hbm.at[wid], idx_v)

    def gather(c, b):
        pltpu.async_copy(table_hbm.at[idx_v.at[c]], rows_v.at[b], gs[b])

    def gwait(c, b):
        pltpu.make_async_copy(table_hbm.at[idx_v.at[c]], rows_v.at[b], gs[b]).wait()

    def store(c, b):
        pltpu.async_copy(
            rows_v.at[b], out_hbm.at[pl.ds(base + c * _CHUNK, _CHUNK)], ss[b]
        )

    def swait(c, b):
        pltpu.make_async_copy(
            rows_v.at[b], out_hbm.at[pl.ds(base + c * _CHUNK, _CHUNK)], ss[b]
        ).wait()

    def slot(c, b, do_swait, do_gather):
        gwait(c, b)
        store(c, b)
        if do_swait:
            swait(c - (_NBUF - _LOOK), (b + _LOOK) % _NBUF)
        if do_gather:
            gather(c + _LOOK, (b + _LOOK) % _NBUF)

    for c in range(_LOOK):
        gather(c, c)
    for c in range(_NBUF - _LOOK):
        slot(c, c, False, True)

    first = _NBUF - _LOOK
    last = _NCHUNK - _LOOK
    nloop = (last - first) // _NBUF

    def body(t, carry):
        c0 = first + t * _NBUF
        for i in range(_NBUF):
            slot(c0 + i, (first + i) % _NBUF, True, True)
        return carry

    lax.fori_loop(0, nloop, body, 0)

    for c in range(first + nloop * _NBUF, last):
        slot(c, c % _NBUF, True, True)

    for c in range(last, _NCHUNK):
        slot(c, c % _NBUF, True, False)
    for c in range(_NCHUNK - (_NBUF - _LOOK), _NCHUNK):
        swait(c, c % _NBUF)


_emb = functools.partial(
    pl.kernel,
    mesh=plsc.VectorSubcoreMesh(core_axis_name="c", subcore_axis_name="s"),
    out_type=jax.ShapeDtypeStruct((_B, _D), jnp.float32),
    scratch_types=[
        pltpu.VMEM((_NCHUNK, _CHUNK), jnp.int32),
        pltpu.VMEM((_NBUF, _CHUNK, _D), jnp.float32),
    ] + [pltpu.SemaphoreType.DMA] * (2 * _NBUF),
)(_emb_body)


@jax.jit
def kernel(token_ids, weight):
    flat = token_ids.reshape(_NW, _NCHUNK, _CHUNK).astype(jnp.int32)
    out = _emb(flat, weight)
    return out.reshape(*token_ids.shape, _D)


# speedup vs baseline: 5.9487x; 1.7880x over previous
import functools

import jax
import jax.numpy as jnp
from jax import lax
from jax.experimental import pallas as pl
from jax.experimental.pallas import tpu as pltpu
from jax.experimental.pallas import tpu_sc as plsc

_D = 128
_S = 4096
_T = 50
_NW = 32
_SPW = _S // _NW
_K = 4
_NSLOT = _SPW // _K
_NBUF = 4
_LOOK = 2


def _emb_body(idx_hbm, table_hbm, out_hbm, idx_v, rows_v, *sems):
    gs = sems[:_NBUF]
    ss = sems[_NBUF:]
    wid = lax.axis_index("s") * 2 + lax.axis_index("c")
    sbase = wid * _SPW

    pltpu.sync_copy(idx_hbm.at[pl.ds(sbase, _SPW)], idx_v)

    def gather(c, b):
        for j in range(_K):
            pltpu.async_copy(
                table_hbm.at[idx_v.at[c * _K + j]],
                rows_v.at[b].at[j],
                gs[b],
            )

    def gwait(c, b):
        for j in range(_K):
            pltpu.make_async_copy(
                table_hbm.at[idx_v.at[c * _K + j]],
                rows_v.at[b].at[j],
                gs[b],
            ).wait()

    def store(c, b):
        pltpu.async_copy(
            rows_v.at[b], out_hbm.at[pl.ds(sbase + c * _K, _K)], ss[b]
        )

    def swait(c, b):
        pltpu.make_async_copy(
            rows_v.at[b], out_hbm.at[pl.ds(sbase + c * _K, _K)], ss[b]
        ).wait()

    def slot(c, b, do_swait, do_gather):
        gwait(c, b)
        store(c, b)
        if do_swait:
            swait(c - (_NBUF - _LOOK), (b + _LOOK) % _NBUF)
        if do_gather:
            gather(c + _LOOK, (b + _LOOK) % _NBUF)

    for c in range(_LOOK):
        gather(c, c)
    for c in range(_NBUF - _LOOK):
        slot(c, c, False, True)

    first = _NBUF - _LOOK
    last = _NSLOT - _LOOK
    nloop = (last - first) // _NBUF

    def body(t, carry):
        c0 = first + t * _NBUF
        for i in range(_NBUF):
            slot(c0 + i, (first + i) % _NBUF, True, True)
        return carry

    lax.fori_loop(0, nloop, body, 0)

    for c in range(first + nloop * _NBUF, last):
        slot(c, c % _NBUF, True, True)

    for c in range(last, _NSLOT):
        slot(c, c % _NBUF, True, False)
    for c in range(_NSLOT - (_NBUF - _LOOK), _NSLOT):
        swait(c, c % _NBUF)


_emb = functools.partial(
    pl.kernel,
    mesh=plsc.VectorSubcoreMesh(core_axis_name="c", subcore_axis_name="s"),
    out_type=jax.ShapeDtypeStruct((_S, _T, _D), jnp.float32),
    scratch_types=[
        pltpu.VMEM((_SPW, _T), jnp.int32),
        pltpu.VMEM((_NBUF, _K, _T, _D), jnp.float32),
    ] + [pltpu.SemaphoreType.DMA] * (2 * _NBUF),
)(_emb_body)


@jax.jit
def kernel(token_ids, weight):
    return _emb(token_ids.astype(jnp.int32), weight)
